# baseline (speedup 1.0000x reference)
import functools
import math

import jax
import jax.numpy as jnp
from jax import lax
from jax.experimental import pallas as pl
from jax.experimental.pallas import tpu as pltpu
from jax.experimental.pallas import tpu_sc as plsc

_NUM_CORES = 2
_NUM_SUBCORES = 16
_NUM_WORKERS = _NUM_CORES * _NUM_SUBCORES
_LANES = 16
_BBLK = 128
_K = 4
_RBLK = 384


def _worker_id():
    return lax.axis_index("s") * _NUM_CORES + lax.axis_index("c")


def _make_prep_kernel(vocab, d):
    scale = math.sqrt(d)
    mesh = plsc.VectorSubcoreMesh(core_axis_name="c", subcore_axis_name="s")
    n_blocks = -(-vocab // _RBLK)
    iters = -(-n_blocks // _NUM_WORKERS)

    @functools.partial(
        pl.kernel,
        out_type=jax.ShapeDtypeStruct((vocab, d), jnp.float32),
        mesh=mesh,
        compiler_params=pltpu.CompilerParams(
            use_tc_tiling_on_sc=False, needs_layout_passes=False
        ),
        scratch_types=[
            pltpu.VMEM((d, _RBLK + 1), jnp.float32),
            pltpu.VMEM((d, _RBLK + 1), jnp.float32),
            pltpu.VMEM((_RBLK, d), jnp.float32),
            pltpu.VMEM((_RBLK, d), jnp.float32),
            pltpu.SemaphoreType.DMA,
            pltpu.SemaphoreType.DMA,
        ],
    )
    def prep(tt_hbm, tr_hbm, ab0, ab1, ob0, ob1, ssem, wsem):
        w = _worker_id()
        lanes = lax.iota(jnp.int32, _LANES)

        def blk_start(it):
            blk = it * _NUM_WORKERS + w
            return jnp.minimum(blk * _RBLK, vocab - _RBLK), blk < n_blocks

        def stage(ab, r0):
            for c in range(d):
                pltpu.async_copy(
                    tt_hbm.at[c, pl.ds(r0, _RBLK)],
                    ab.at[c, pl.ds(0, _RBLK)],
                    ssem,
                )

        def stage_wait(ab):
            for c in range(d):
                pltpu.make_async_copy(
                    tt_hbm.at[0, pl.ds(0, _RBLK)], ab.at[c, pl.ds(0, _RBLK)], ssem
                ).wait()

        def transpose(ab, ob):
            def rloop(r, carry):
                rv = jnp.full((_LANES,), 0, jnp.int32) + r
                for g in range(d // _LANES):
                    vals = plsc.load_gather(ab, [g * _LANES + lanes, rv])
                    ob[r, pl.ds(g * _LANES, _LANES)] = vals * scale
                return carry

            lax.fori_loop(0, _RBLK, rloop, 0, unroll=2)

        bufs = ((ab0, ob0), (ab1, ob1))
        r0_first, _ = blk_start(0)
        stage(ab0, r0_first)

        def body(i, carry):
            for par in range(2):
                it = 2 * i + par
                ab, ob = bufs[par]
                nab, _nob = bufs[1 - par]
                r0, valid = blk_start(it)
                nr0, nvalid = blk_start(it + 1)

                @pl.when(valid)
                def _():
                    stage_wait(ab)

                    @pl.when(nvalid)
                    def _():
                        stage(nab, nr0)

                    transpose(ab, ob)

                    @pl.when((it >= 2) & (((it - 2) * _NUM_WORKERS + w) < n_blocks))
                    def _():
                        pltpu.make_async_copy(
                            tr_hbm.at[pl.ds(0, _RBLK)], ob, wsem
                        ).wait()

                    pltpu.async_copy(ob, tr_hbm.at[pl.ds(r0, _RBLK)], wsem)

            return carry

        lax.fori_loop(0, -(-iters // 2), body, 0)
        for par in range(2):
            pltpu.make_async_copy(
                tr_hbm.at[pl.ds(0, _RBLK)], bufs[par][1], wsem
            ).wait()

    return prep


def _make_gather_kernel(batch, seq, d):
    mesh = plsc.VectorSubcoreMesh(core_axis_name="c", subcore_axis_name="s")

    gbufs = [pltpu.VMEM((_BBLK, d), jnp.float32) for _ in range(_K)]
    tbufs = [pltpu.VMEM((d // 8, 8, _BBLK + 1), jnp.float32) for _ in range(_K)]

    @functools.partial(
        pl.kernel,
        out_type=jax.ShapeDtypeStruct(
            (seq, d // 8, batch // _BBLK, 8, _BBLK), jnp.float32
        ),
        mesh=mesh,
        compiler_params=pltpu.CompilerParams(
            use_tc_tiling_on_sc=False, needs_layout_passes=False
        ),
        scratch_types=[pltpu.VMEM((seq // 8, 8, _BBLK), jnp.int32)]
        + gbufs
        + tbufs
        + [pltpu.SemaphoreType.DMA, pltpu.SemaphoreType.DMA],
    )
    def emb(tr_hbm, idx_hbm, q_hbm, idx_v, *bufs_and_sems):
        gb = bufs_and_sems[:_K]
        tb = bufs_and_sems[_K : 2 * _K]
        gsem, wsem = bufs_and_sems[2 * _K], bufs_and_sems[2 * _K + 1]
        w = _worker_id()
        pltpu.sync_copy(idx_hbm.at[:, w], idx_v)
        lanes = lax.iota(jnp.int32, _LANES)
        i0s = [(g * _LANES + lanes) >> 3 for g in range(d // _LANES)]
        i1s = [(g * _LANES + lanes) & 7 for g in range(d // _LANES)]

        def body(i, carry):
            s0 = i * _K
            handles = [
                pltpu.async_copy(
                    tr_hbm.at[idx_v.at[(s0 + k) // 8, (s0 + k) % 8]],
                    gb[k],
                    gsem,
                )
                for k in range(_K)
            ]
            for k in range(_K):
                handles[k].wait()

                def bloop(b, carry2, g=gb[k], t=tb[k]):
                    bv = jnp.full((_LANES,), 0, jnp.int32) + b
                    for grp in range(d // _LANES):
                        vals = g[b, pl.ds(grp * _LANES, _LANES)]
                        plsc.store_scatter(t, [i0s[grp], i1s[grp], bv], vals)
                    return carry2

                lax.fori_loop(0, _BBLK, bloop, 0, unroll=2)

                @pl.when(i > 0)
                def _(t=tb[k]):
                    pltpu.make_async_copy(
                        q_hbm.at[0, :, w], t.at[:, :, pl.ds(0, _BBLK)], wsem
                    ).wait()

                pltpu.async_copy(
                    tb[k].at[:, :, pl.ds(0, _BBLK)], q_hbm.at[s0 + k, :, w], wsem
                )
            return carry

        lax.fori_loop(0, seq // _K, body, 0)
        for k in range(_K):
            pltpu.make_async_copy(
                q_hbm.at[0, :, w], tb[k].at[:, :, pl.ds(0, _BBLK)], wsem
            ).wait()

    return emb


def kernel(x, table):
    batch, seq = x.shape
    vocab, d = table.shape
    table_t = table.T
    x4 = (
        x.T.astype(jnp.int32)
        .reshape(seq // 8, 8, batch // _BBLK, _BBLK)
        .transpose(0, 2, 1, 3)
    )
    prep = _make_prep_kernel(vocab, d)
    table_r = prep(table_t)
    emb = _make_gather_kernel(batch, seq, d)
    q = emb(table_r, x4)
    out = q.transpose(2, 4, 0, 1, 3).reshape(batch, seq, d)
    return out


# speedup vs baseline: 3.7762x; 3.7762x over previous
import functools
import math

import jax
import jax.numpy as jnp
from jax import lax
from jax.experimental import pallas as pl
from jax.experimental.pallas import tpu as pltpu
from jax.experimental.pallas import tpu_sc as plsc

_NUM_CORES = 2
_NUM_SUBCORES = 16
_NUM_WORKERS = _NUM_CORES * _NUM_SUBCORES
_LANES = 16
_BBLK = 128
_K = 4
_RBLK = 384


def _worker_id():
    return lax.axis_index("s") * _NUM_CORES + lax.axis_index("c")


def _make_prep_kernel(vocab, d):
    scale = math.sqrt(d)
    rblk = 512
    grid = -(-vocab // rblk)

    def body(tt_ref, out_ref):
        out_ref[:, 0:d] = tt_ref[...].T * scale

    return pl.pallas_call(
        body,
        grid=(grid,),
        in_specs=[pl.BlockSpec((d, rblk), lambda i: (0, i))],
        out_specs=pl.BlockSpec((rblk, 2 * d), lambda i: (i, 0)),
        out_shape=jax.ShapeDtypeStruct((vocab, 2 * d), jnp.float32),
    )


def _make_gather_kernel(batch, seq, d):
    mesh = plsc.VectorSubcoreMesh(core_axis_name="c", subcore_axis_name="s")

    gbufs = [pltpu.VMEM((_BBLK, d), jnp.float32) for _ in range(_K)]
    tbufs = [pltpu.VMEM((d // 8, 8, _BBLK + 1), jnp.float32) for _ in range(_K)]

    @functools.partial(
        pl.kernel,
        out_type=jax.ShapeDtypeStruct(
            (seq, d // 8, batch // _BBLK, 8, _BBLK), jnp.float32
        ),
        mesh=mesh,
        compiler_params=pltpu.CompilerParams(
            use_tc_tiling_on_sc=False, needs_layout_passes=False
        ),
        scratch_types=[pltpu.VMEM((seq // 8, 8, _BBLK), jnp.int32)]
        + gbufs
        + tbufs
        + [pltpu.SemaphoreType.DMA, pltpu.SemaphoreType.DMA],
    )
    def emb(tr_hbm, idx_hbm, q_hbm, idx_v, *bufs_and_sems):
        gb = bufs_and_sems[:_K]
        tb = bufs_and_sems[_K : 2 * _K]
        gsem, wsem = bufs_and_sems[2 * _K], bufs_and_sems[2 * _K + 1]
        w = _worker_id()
        pltpu.sync_copy(idx_hbm.at[:, w], idx_v)
        lanes = lax.iota(jnp.int32, _LANES)
        i0s = [(g * _LANES + lanes) >> 3 for g in range(d // _LANES)]
        i1s = [(g * _LANES + lanes) & 7 for g in range(d // _LANES)]

        def body(i, carry):
            s0 = i * _K
            handles = [
                pltpu.async_copy(
                    tr_hbm.at[idx_v.at[(s0 + k) // 8, (s0 + k) % 8]],
                    gb[k],
                    gsem,
                )
                for k in range(_K)
            ]
            for k in range(_K):
                handles[k].wait()

                def bloop(b, carry2, g=gb[k], t=tb[k]):
                    bv = jnp.full((_LANES,), 0, jnp.int32) + b
                    for grp in range(d // _LANES):
                        vals = g[b, pl.ds(grp * _LANES, _LANES)]
                        plsc.store_scatter(t, [i0s[grp], i1s[grp], bv], vals)
                    return carry2

                lax.fori_loop(0, _BBLK, bloop, 0, unroll=2)

                @pl.when(i > 0)
                def _(t=tb[k]):
                    pltpu.make_async_copy(
                        q_hbm.at[0, :, w], t.at[:, :, pl.ds(0, _BBLK)], wsem
                    ).wait()

                pltpu.async_copy(
                    tb[k].at[:, :, pl.ds(0, _BBLK)], q_hbm.at[s0 + k, :, w], wsem
                )
            return carry

        lax.fori_loop(0, seq // _K, body, 0)
        for k in range(_K):
            pltpu.make_async_copy(
                q_hbm.at[0, :, w], tb[k].at[:, :, pl.ds(0, _BBLK)], wsem
            ).wait()

    return emb


def kernel(x, table):
    batch, seq = x.shape
    vocab, d = table.shape
    table_t = table.T
    x4 = (
        x.T.astype(jnp.int32)
        .reshape(seq // 8, 8, batch // _BBLK, _BBLK)
        .transpose(0, 2, 1, 3)
    ) * 2
    prep = _make_prep_kernel(vocab, d)
    table_r = prep(table_t).reshape(2 * vocab, d)
    emb = _make_gather_kernel(batch, seq, d)
    q = emb(table_r, x4)
    out = q.transpose(2, 4, 0, 1, 3).reshape(batch, seq, d)
    return out


# speedup vs baseline: 6.7033x; 1.7752x over previous
import functools
import math

import jax
import jax.numpy as jnp
from jax import lax
from jax.experimental import pallas as pl
from jax.experimental.pallas import tpu as pltpu
from jax.experimental.pallas import tpu_sc as plsc

_NUM_CORES = 2
_NUM_SUBCORES = 16
_NUM_WORKERS = _NUM_CORES * _NUM_SUBCORES
_LANES = 16
_BBLK = 128
_K = 4
_RBLK = 384


def _worker_id():
    return lax.axis_index("s") * _NUM_CORES + lax.axis_index("c")


def _make_prep_kernel(vocab, d):
    rblk = 2048
    grid = -(-vocab // rblk)

    def body(tt_ref, eye_ref, out_ref):
        out_ref[:, 0:d] = lax.dot_general(
            tt_ref[...],
            eye_ref[...],
            (((0,), (0,)), ((), ())),
            preferred_element_type=jnp.float32,
        )

    return pl.pallas_call(
        body,
        grid=(grid,),
        in_specs=[
            pl.BlockSpec((d, rblk), lambda i: (0, i)),
            pl.BlockSpec((d, d), lambda i: (0, 0)),
        ],
        out_specs=pl.BlockSpec((rblk, 2 * d), lambda i: (i, 0)),
        out_shape=jax.ShapeDtypeStruct((vocab, 2 * d), jnp.float32),
    )


def _make_gather_kernel(batch, seq, d):
    mesh = plsc.VectorSubcoreMesh(core_axis_name="c", subcore_axis_name="s")

    gbufs = [pltpu.VMEM((_BBLK, d), jnp.float32) for _ in range(_K)]
    tbufs = [pltpu.VMEM((d // 8, 8, _BBLK + 1), jnp.float32) for _ in range(_K)]

    @functools.partial(
        pl.kernel,
        out_type=jax.ShapeDtypeStruct(
            (seq, d // 8, batch // _BBLK, 8, _BBLK), jnp.float32
        ),
        mesh=mesh,
        compiler_params=pltpu.CompilerParams(
            use_tc_tiling_on_sc=False, needs_layout_passes=False
        ),
        scratch_types=[pltpu.VMEM((seq // 8, 8, _BBLK), jnp.int32)]
        + gbufs
        + tbufs
        + [pltpu.SemaphoreType.DMA, pltpu.SemaphoreType.DMA],
    )
    def emb(tr_hbm, idx_hbm, q_hbm, idx_v, *bufs_and_sems):
        gb = bufs_and_sems[:_K]
        tb = bufs_and_sems[_K : 2 * _K]
        gsem, wsem = bufs_and_sems[2 * _K], bufs_and_sems[2 * _K + 1]
        w = _worker_id()
        pltpu.sync_copy(idx_hbm.at[:, w], idx_v)
        lanes = lax.iota(jnp.int32, _LANES)
        i0s = [(g * _LANES + lanes) >> 3 for g in range(d // _LANES)]
        i1s = [(g * _LANES + lanes) & 7 for g in range(d // _LANES)]

        def body(i, carry):
            s0 = i * _K
            handles = [
                pltpu.async_copy(
                    tr_hbm.at[idx_v.at[(s0 + k) // 8, (s0 + k) % 8]],
                    gb[k],
                    gsem,
                )
                for k in range(_K)
            ]
            for k in range(_K):
                handles[k].wait()

                def bloop(b, carry2, g=gb[k], t=tb[k]):
                    bv = jnp.full((_LANES,), 0, jnp.int32) + b
                    for grp in range(d // _LANES):
                        vals = g[b, pl.ds(grp * _LANES, _LANES)]
                        plsc.store_scatter(t, [i0s[grp], i1s[grp], bv], vals)
                    return carry2

                lax.fori_loop(0, _BBLK, bloop, 0, unroll=8)

                @pl.when(i > 0)
                def _(t=tb[k]):
                    pltpu.make_async_copy(
                        q_hbm.at[0, :, w], t.at[:, :, pl.ds(0, _BBLK)], wsem
                    ).wait()

                pltpu.async_copy(
                    tb[k].at[:, :, pl.ds(0, _BBLK)], q_hbm.at[s0 + k, :, w], wsem
                )
            return carry

        lax.fori_loop(0, seq // _K, body, 0)
        for k in range(_K):
            pltpu.make_async_copy(
                q_hbm.at[0, :, w], tb[k].at[:, :, pl.ds(0, _BBLK)], wsem
            ).wait()

    return emb


def kernel(x, table):
    batch, seq = x.shape
    vocab, d = table.shape
    table_t = table.T
    x4 = (
        x.T.astype(jnp.int32)
        .reshape(seq // 8, 8, batch // _BBLK, _BBLK)
        .transpose(0, 2, 1, 3)
    ) * 2
    prep = _make_prep_kernel(vocab, d)
    eye = jnp.eye(d, dtype=jnp.float32) * math.sqrt(d)
    table_r = prep(table_t, eye).reshape(2 * vocab, d)
    emb = _make_gather_kernel(batch, seq, d)
    q = emb(table_r, x4)
    out = q.transpose(2, 4, 0, 1, 3).reshape(batch, seq, d)
    return out
